# baseline (speedup 1.0000x reference)
import jax
import jax.numpy as jnp
from jax.experimental import pallas as pl
from jax.experimental.pallas import tpu as pltpu

_N = 10000
_F = 128
_BM = 200
_NBUF = 4


def _gcn_kernel(a_ref, seq_ref, w_ref, adj_hbm, out_ref, h_ref, bufs, sems):
    i = pl.program_id(0)
    steps = pl.num_programs(0)

    @pl.when(i == 0)
    def _prologue():
        h_ref[...] = jax.lax.dot_general(
            seq_ref[...], w_ref[...],
            dimension_numbers=(((1,), (1,)), ((), ())),
            preferred_element_type=jnp.float32)
        for b in range(_NBUF - 1):
            pltpu.make_async_copy(
                adj_hbm.at[pl.ds(b * _BM, _BM), :],
                bufs.at[b], sems.at[b]).start()

    nxt = i + _NBUF - 1
    nxt_slot = jax.lax.rem(nxt, _NBUF)
    for b in range(_NBUF):
        @pl.when(jnp.logical_and(nxt < steps, nxt_slot == b))
        def _prefetch(b=b):
            pltpu.make_async_copy(
                adj_hbm.at[pl.ds(nxt * _BM, _BM), :],
                bufs.at[b], sems.at[b]).start()

    slot = jax.lax.rem(i, _NBUF)
    for b in range(_NBUF):
        @pl.when(slot == b)
        def _compute(b=b):
            pltpu.make_async_copy(
                adj_hbm.at[pl.ds(i * _BM, _BM), :],
                bufs.at[b], sems.at[b]).wait()
            acc = jnp.dot(bufs[b], h_ref[...],
                          preferred_element_type=jnp.float32)
            a = a_ref[0]
            out_ref[...] = jnp.where(acc > 0, acc, a * acc)


def kernel(seq, adj, W, prelu_a):
    seq2 = seq.reshape(_N, _F)
    adj2 = adj.reshape(_N, _N)

    out = pl.pallas_call(
        _gcn_kernel,
        grid=(_N // _BM,),
        in_specs=[
            pl.BlockSpec(memory_space=pltpu.SMEM),
            pl.BlockSpec((_N, _F), lambda i: (0, 0)),
            pl.BlockSpec((_F, _F), lambda i: (0, 0)),
            pl.BlockSpec(memory_space=pl.ANY),
        ],
        out_specs=pl.BlockSpec((_BM, _F), lambda i: (i, 0)),
        out_shape=jax.ShapeDtypeStruct((_N, _F), jnp.float32),
        scratch_shapes=[
            pltpu.VMEM((_N, _F), jnp.float32),
            pltpu.VMEM((_NBUF, _BM, _N), jnp.float32),
            pltpu.SemaphoreType.DMA((_NBUF,)),
        ],
    )(prelu_a, seq2, W, adj2)

    return out.reshape(1, _N, _F)


# speedup vs baseline: 1.0270x; 1.0270x over previous
import jax
import jax.numpy as jnp
from jax.experimental import pallas as pl
from jax.experimental.pallas import tpu as pltpu

_N = 10000
_F = 128
_BM = 240


def _gcn_kernel(a_ref, seq_ref, w_ref, adj_ref, out_ref, h_ref):
    @pl.when(pl.program_id(0) == 0)
    def _compute_h():
        h_ref[...] = jax.lax.dot_general(
            seq_ref[...], w_ref[...],
            dimension_numbers=(((1,), (1,)), ((), ())),
            preferred_element_type=jnp.float32)

    acc = jnp.dot(adj_ref[...], h_ref[...], preferred_element_type=jnp.float32)
    a = a_ref[0]
    out_ref[...] = jnp.where(acc > 0, acc, a * acc)


def kernel(seq, adj, W, prelu_a):
    seq2 = seq.reshape(_N, _F)
    adj2 = adj.reshape(_N, _N)

    out = pl.pallas_call(
        _gcn_kernel,
        grid=(pl.cdiv(_N, _BM),),
        in_specs=[
            pl.BlockSpec(memory_space=pltpu.SMEM),
            pl.BlockSpec((_N, _F), lambda i: (0, 0)),
            pl.BlockSpec((_F, _F), lambda i: (0, 0)),
            pl.BlockSpec((_BM, _N), lambda i: (i, 0)),
        ],
        out_specs=pl.BlockSpec((_BM, _F), lambda i: (i, 0)),
        out_shape=jax.ShapeDtypeStruct((_N, _F), jnp.float32),
        scratch_shapes=[pltpu.VMEM((_N, _F), jnp.float32)],
    )(prelu_a, seq2, W, adj2)

    return out.reshape(1, _N, _F)


# speedup vs baseline: 1.0301x; 1.0030x over previous
import jax
import jax.numpy as jnp
from jax.experimental import pallas as pl
from jax.experimental.pallas import tpu as pltpu

_N = 10000
_F = 128
_BM = 240


def _gcn_kernel(a_ref, seq_ref, w_ref, adj_ref, out_ref, h_ref):
    @pl.when(pl.program_id(0) == 0)
    def _compute_h():
        h_ref[...] = jax.lax.dot_general(
            seq_ref[...], w_ref[...],
            dimension_numbers=(((1,), (1,)), ((), ())),
            preferred_element_type=jnp.float32).astype(jnp.bfloat16)

    acc = jax.lax.dot_general(
        adj_ref[...], h_ref[...],
        dimension_numbers=(((1,), (0,)), ((), ())),
        preferred_element_type=jnp.float32)
    a = a_ref[0]
    out_ref[...] = jnp.where(acc > 0, acc, a * acc)


def kernel(seq, adj, W, prelu_a):
    seq2 = seq.reshape(_N, _F)
    adj2 = adj.reshape(_N, _N)

    out = pl.pallas_call(
        _gcn_kernel,
        grid=(pl.cdiv(_N, _BM),),
        in_specs=[
            pl.BlockSpec(memory_space=pltpu.SMEM),
            pl.BlockSpec((_N, _F), lambda i: (0, 0)),
            pl.BlockSpec((_F, _F), lambda i: (0, 0)),
            pl.BlockSpec((_BM, _N), lambda i: (i, 0)),
        ],
        out_specs=pl.BlockSpec((_BM, _F), lambda i: (i, 0)),
        out_shape=jax.ShapeDtypeStruct((_N, _F), jnp.float32),
        scratch_shapes=[pltpu.VMEM((_N, _F), jnp.bfloat16)],
    )(prelu_a, seq2, W, adj2)

    return out.reshape(1, _N, _F)
